# baseline (speedup 1.0000x reference)
import jax
import jax.numpy as jnp
from jax.experimental import pallas as pl

C = 96
H = 512
W = 512
R = 16
NB = H // R
BL = R * W


def _conv_block(x34, wt, tap0):
    acc = None
    for kh in range(3):
        wk = wt[(kh * 3 + tap0) * C:(kh * 3 + tap0) * C + C, :]
        xk = x34[:, kh * W:kh * W + BL]
        d = jax.lax.dot_general(wk, xk, (((1,), (0,)), ((), ())),
                                preferred_element_type=jnp.float32)
        acc = d if acc is None else acc + d
    return acc


def _kernel(up_ref, x_ref, dn_ref, wt_ref, spa_ref, m0_ref, m1_ref,
            a_ref, dmask_ref, b_ref, out_ref):
    up = up_ref[0]
    dn = dn_ref[0]
    x34 = jnp.concatenate([up, x_ref[...], dn], axis=1).astype(jnp.bfloat16)
    wt = wt_ref[...].astype(jnp.bfloat16)

    acc = _conv_block(x34, wt, 1)
    t0 = _conv_block(x34, wt, 0)
    z = jnp.zeros((C, 1), dtype=jnp.float32)
    sr = jnp.concatenate([z, t0[:, :-1]], axis=1)
    acc = acc + sr * m0_ref[...]
    t2 = _conv_block(x34, wt, 2)
    sl = jnp.concatenate([t2[:, 1:], z], axis=1)
    acc = acc + sl * m1_ref[...]

    fea = acc + b_ref[...]
    scale = a_ref[...] * spa_ref[...] + dmask_ref[...]
    out_ref[...] = jnp.maximum(fea * scale, 0.0)


def kernel(x0, spa_mask, Wc, b, ch_mask):
    u = jax.random.uniform(jax.random.key(1234), ch_mask.shape,
                           minval=1e-8, maxval=1.0 - 1e-8)
    g = -jnp.log(-jnp.log(u))
    cm = jax.nn.softmax((ch_mask + g) / 1.0, axis=-1)

    x2 = x0.reshape(C, H, W)
    xf = x0.reshape(C, H * W)
    spa = spa_mask.reshape(1, H * W)

    zrow = jnp.zeros((1, C, W), dtype=jnp.float32)
    ups = jnp.concatenate(
        [zrow, jnp.transpose(x2[:, R - 1:H - R:R, :], (1, 0, 2))], axis=0)
    dns = jnp.concatenate(
        [jnp.transpose(x2[:, R:H:R, :], (1, 0, 2)), zrow], axis=0)

    wt = jnp.transpose(Wc, (2, 3, 0, 1)).reshape(9 * C, C)

    col = jnp.arange(H * W, dtype=jnp.int32) % W
    m0 = (col != 0).astype(jnp.float32).reshape(1, H * W)
    m1 = (col != W - 1).astype(jnp.float32).reshape(1, H * W)

    a = cm[0, :, 0].reshape(C, 1)
    dm = cm[0, :, 1].reshape(C, 1)
    bb = b.reshape(C, 1)

    out = pl.pallas_call(
        _kernel,
        grid=(NB,),
        in_specs=[
            pl.BlockSpec((1, C, W), lambda i: (i, 0, 0)),
            pl.BlockSpec((C, BL), lambda i: (0, i)),
            pl.BlockSpec((1, C, W), lambda i: (i, 0, 0)),
            pl.BlockSpec((9 * C, C), lambda i: (0, 0)),
            pl.BlockSpec((1, BL), lambda i: (0, i)),
            pl.BlockSpec((1, BL), lambda i: (0, i)),
            pl.BlockSpec((1, BL), lambda i: (0, i)),
            pl.BlockSpec((C, 1), lambda i: (0, 0)),
            pl.BlockSpec((C, 1), lambda i: (0, 0)),
            pl.BlockSpec((C, 1), lambda i: (0, 0)),
        ],
        out_specs=pl.BlockSpec((C, BL), lambda i: (0, i)),
        out_shape=jax.ShapeDtypeStruct((C, H * W), jnp.float32),
    )(ups, xf, dns, wt, spa, m0, m1, a, dm, bb)

    return (out.reshape(1, C, H, W), cm)


# speedup vs baseline: 1.2498x; 1.2498x over previous
import jax
import jax.numpy as jnp
from jax.experimental import pallas as pl

C = 96
H = 512
W = 512
R = 16
NB = H // R
BL = R * W


def _conv_block(x34, wt, tap0):
    acc = None
    for kh in range(3):
        wk = wt[(kh * 3 + tap0) * C:(kh * 3 + tap0) * C + C, :]
        xk = x34[:, kh * W:kh * W + BL]
        d = jax.lax.dot_general(wk, xk, (((1,), (0,)), ((), ())),
                                preferred_element_type=jnp.float32)
        acc = d if acc is None else acc + d
    return acc


def _kernel(up_ref, x_ref, dn_ref, wt_ref, spa_ref, a_ref, dmask_ref, b_ref,
            out_ref):
    i = pl.program_id(0)
    up = up_ref[...] * jnp.where(i == 0, 0.0, 1.0)
    dn = dn_ref[...] * jnp.where(i == NB - 1, 0.0, 1.0)
    x34 = jnp.concatenate([up, x_ref[...], dn], axis=1).astype(jnp.bfloat16)
    wt = wt_ref[...].astype(jnp.bfloat16)

    col = jax.lax.broadcasted_iota(jnp.int32, (1, BL), 1) % W
    m0 = (col != 0).astype(jnp.float32)
    m1 = (col != W - 1).astype(jnp.float32)

    acc = _conv_block(x34, wt, 1)
    t0 = _conv_block(x34, wt, 0)
    z = jnp.zeros((C, 1), dtype=jnp.float32)
    sr = jnp.concatenate([z, t0[:, :-1]], axis=1)
    acc = acc + sr * m0
    t2 = _conv_block(x34, wt, 2)
    sl = jnp.concatenate([t2[:, 1:], z], axis=1)
    acc = acc + sl * m1

    fea = acc + b_ref[...]
    scale = a_ref[...] * spa_ref[...] + dmask_ref[...]
    out_ref[...] = jnp.maximum(fea * scale, 0.0)


def kernel(x0, spa_mask, Wc, b, ch_mask):
    u = jax.random.uniform(jax.random.key(1234), ch_mask.shape,
                           minval=1e-8, maxval=1.0 - 1e-8)
    g = -jnp.log(-jnp.log(u))
    cm = jax.nn.softmax((ch_mask + g) / 1.0, axis=-1)

    xf = x0.reshape(C, H * W)
    spa = spa_mask.reshape(1, H * W)

    wt = jnp.transpose(Wc, (2, 3, 0, 1)).reshape(9 * C, C)

    a = cm[0, :, 0].reshape(C, 1)
    dm = cm[0, :, 1].reshape(C, 1)
    bb = b.reshape(C, 1)

    out = pl.pallas_call(
        _kernel,
        grid=(NB,),
        in_specs=[
            pl.BlockSpec((C, W), lambda i: (0, jnp.maximum(i * R - 1, 0))),
            pl.BlockSpec((C, BL), lambda i: (0, i)),
            pl.BlockSpec((C, W), lambda i: (0, jnp.minimum(i * R + R, H - 1))),
            pl.BlockSpec((9 * C, C), lambda i: (0, 0)),
            pl.BlockSpec((1, BL), lambda i: (0, i)),
            pl.BlockSpec((C, 1), lambda i: (0, 0)),
            pl.BlockSpec((C, 1), lambda i: (0, 0)),
            pl.BlockSpec((C, 1), lambda i: (0, 0)),
        ],
        out_specs=pl.BlockSpec((C, BL), lambda i: (0, i)),
        out_shape=jax.ShapeDtypeStruct((C, H * W), jnp.float32),
    )(xf, xf, xf, wt, spa, a, dm, bb)

    return (out.reshape(1, C, H, W), cm)
